# Initial kernel scaffold; baseline (speedup 1.0000x reference)
import jax
import jax.numpy as jnp
from jax.experimental import pallas as pl


def kernel(query0, query1, query2, feat0, feat1, feat2, w_value, b_value, w_off, b_off, w_attn, b_attn, w_out, b_out):
    raise NotImplementedError("write your pallas kernel here")


# speedup vs baseline: 5003.8294x; 5003.8294x over previous
import functools

import jax
import jax.numpy as jnp
import numpy as np
from jax import lax
from jax.experimental import pallas as pl
from jax.experimental.pallas import tpu as pltpu
from jax.experimental.pallas import tpu_sc as plsc

DIM = 256
NH = 8
NL = 3
NP = 4
DH = DIM // NH
BATCH = 4
SHAPES = [(64, 96), (32, 48), (16, 24)]
SIZES = [h * w for (h, w) in SHAPES]
LQ = sum(SIZES)
LVL_BASE = [0, SIZES[0], SIZES[0] + SIZES[1]]
PAD = 128
NROWS = BATCH * NH * LQ
NCORN = NL * NP * 4

BLKQ = 1152
NQB = LQ // BLKQ

NWORK = 32
RPW = NROWS // NWORK
SB = 128
NSBS = RPW // SB
CHUNK_ROWS = 2
NCH = SB // CHUNK_ROWS


def _pos_embed_levels():
    num_pos_feats = DIM // 2
    temperature = 10000.0
    dim_t = temperature ** (2.0 * np.floor(np.arange(num_pos_feats) / 2.0) / num_pos_feats)
    outs = []
    for (h, w) in SHAPES:
        y_embed = np.arange(1, h + 1, dtype=np.float64)[:, None] * np.ones((1, w))
        x_embed = np.arange(1, w + 1, dtype=np.float64)[None, :] * np.ones((h, 1))
        pos_x = x_embed[..., None] / dim_t
        pos_y = y_embed[..., None] / dim_t
        px = np.stack([np.sin(pos_x[..., 0::2]), np.cos(pos_x[..., 1::2])], axis=3).reshape(h, w, -1)
        py = np.stack([np.sin(pos_y[..., 0::2]), np.cos(pos_y[..., 1::2])], axis=3).reshape(h, w, -1)
        pos = np.concatenate([py, px], axis=2)
        outs.append(pos.reshape(h * w, DIM).T.astype(np.float32))
    return np.concatenate(outs, axis=1)


def _ref_scaled():
    refx, refy = [], []
    for (h, w) in SHAPES:
        ry = (np.arange(h) + 0.5) / h
        rx = (np.arange(w) + 0.5) / w
        gy, gx = np.meshgrid(ry, rx, indexing="ij")
        refx.append(gx.reshape(-1))
        refy.append(gy.reshape(-1))
    refx = np.concatenate(refx)
    refy = np.concatenate(refy)
    rows = []
    for (h, w) in SHAPES:
        rows.append(refx * w)
        rows.append(refy * h)
    return np.stack(rows, axis=0).astype(np.float32)


_POS_T = _pos_embed_levels()
_RS = _ref_scaled()
_PERM = np.array(
    [((h * NL + l) * NP + p) * 2 + xy for xy in (0, 1) for h in range(NH) for l in range(NL) for p in range(NP)],
    dtype=np.int32,
)


def _value_proj_body(inp_ref, w_ref, b_ref, out_ref):
    v = jnp.dot(inp_ref[0], w_ref[...], preferred_element_type=jnp.float32)
    v = v + b_ref[...]
    for h in range(NH):
        out_ref[0, h] = v[:, h * DH:(h + 1) * DH]


def _value_proj(inp_rows, w_value, b_value):
    return pl.pallas_call(
        _value_proj_body,
        grid=(BATCH, NQB),
        in_specs=[
            pl.BlockSpec((1, BLKQ, DIM), lambda b, i: (b, i, 0)),
            pl.BlockSpec((DIM, DIM), lambda b, i: (0, 0)),
            pl.BlockSpec((1, DIM), lambda b, i: (0, 0)),
        ],
        out_specs=pl.BlockSpec((1, NH, BLKQ, DH), lambda b, i: (b, 0, i, 0)),
        out_shape=jax.ShapeDtypeStruct((BATCH, NH, LQ, DH), jnp.float32),
        compiler_params=pltpu.CompilerParams(
            dimension_semantics=("parallel", "parallel")),
    )(inp_rows, w_value, b_value[None, :])


def _idx_wts_body(qt_ref, pos_ref, wofft_ref, bofft_ref, wattnt_ref, battnt_ref,
                  rs_ref, idx_ref, wts_ref):
    q = qt_ref[0] + pos_ref[...]
    offt = jnp.dot(wofft_ref[...], q, preferred_element_type=jnp.float32) + bofft_ref[...]
    att = jnp.dot(wattnt_ref[...], q, preferred_element_type=jnp.float32) + battnt_ref[...]
    a3 = att.reshape(NH, NL * NP, BLKQ)
    m = jnp.max(a3, axis=1, keepdims=True)
    e = jnp.exp(a3 - m)
    aw = e / jnp.sum(e, axis=1, keepdims=True)
    offx = offt[0:NH * NL * NP].reshape(NH, NL, NP, BLKQ)
    offy = offt[NH * NL * NP:].reshape(NH, NL, NP, BLKQ)
    pb = pl.program_id(0)
    ih = lax.broadcasted_iota(jnp.int32, (NH, NP, BLKQ), 0)
    idx_parts, wt_parts = [], []
    for l in range(NL):
        hl, wl = SHAPES[l]
        gx = rs_ref[2 * l].reshape(1, 1, BLKQ) + offx[:, l] - 0.5
        gy = rs_ref[2 * l + 1].reshape(1, 1, BLKQ) + offy[:, l] - 0.5
        x0 = jnp.floor(gx)
        y0 = jnp.floor(gy)
        fx = gx - x0
        fy = gy - y0
        vx0 = ((x0 >= 0) & (x0 <= wl - 1)).astype(jnp.float32)
        vx1 = ((x0 >= -1) & (x0 <= wl - 2)).astype(jnp.float32)
        vy0 = ((y0 >= 0) & (y0 <= hl - 1)).astype(jnp.float32)
        vy1 = ((y0 >= -1) & (y0 <= hl - 2)).astype(jnp.float32)
        xi0 = jnp.clip(x0, -1.0, float(wl)).astype(jnp.int32)
        xi1 = jnp.clip(x0 + 1.0, -1.0, float(wl)).astype(jnp.int32)
        yi0 = jnp.clip(y0, -1.0, float(hl)).astype(jnp.int32)
        yi1 = jnp.clip(y0 + 1.0, -1.0, float(hl)).astype(jnp.int32)
        awl = aw[:, l * NP:(l + 1) * NP]
        base = (pb * NH + ih) * LQ + (LVL_BASE[l] + PAD)
        wx0 = (1.0 - fx) * vx0
        wx1 = fx * vx1
        wy0 = (1.0 - fy) * vy0
        wy1 = fy * vy1
        row0 = base + yi0 * wl
        row1 = base + yi1 * wl
        idx_parts.append(jnp.stack(
            [row0 + xi0, row0 + xi1, row1 + xi0, row1 + xi1], axis=2
        ).reshape(NH, NP * 4, BLKQ))
        wt_parts.append(jnp.stack(
            [wy0 * wx0 * awl, wy0 * wx1 * awl, wy1 * wx0 * awl, wy1 * wx1 * awl],
            axis=2).reshape(NH, NP * 4, BLKQ))
    idx = jnp.concatenate(idx_parts, axis=1).reshape(NH * NCORN, BLKQ)
    wts = jnp.concatenate(wt_parts, axis=1).reshape(NH * NCORN, BLKQ)
    idx_ref[0] = idx.T
    wts_ref[0] = wts.T


def _idx_wts(q_t, w_off, b_off, w_attn, b_attn):
    wofft = jnp.transpose(w_off[:, _PERM])
    bofft = b_off[_PERM][:, None]
    wattnt = jnp.transpose(w_attn)
    battnt = b_attn[:, None]
    return pl.pallas_call(
        _idx_wts_body,
        grid=(BATCH, NQB),
        in_specs=[
            pl.BlockSpec((1, DIM, BLKQ), lambda b, i: (b, 0, i)),
            pl.BlockSpec((DIM, BLKQ), lambda b, i: (0, i)),
            pl.BlockSpec((NH * NL * NP * 2, DIM), lambda b, i: (0, 0)),
            pl.BlockSpec((NH * NL * NP * 2, 1), lambda b, i: (0, 0)),
            pl.BlockSpec((NH * NL * NP, DIM), lambda b, i: (0, 0)),
            pl.BlockSpec((NH * NL * NP, 1), lambda b, i: (0, 0)),
            pl.BlockSpec((2 * NL, BLKQ), lambda b, i: (0, i)),
        ],
        out_specs=[
            pl.BlockSpec((1, BLKQ, NH * NCORN), lambda b, i: (b, i, 0)),
            pl.BlockSpec((1, BLKQ, NH * NCORN), lambda b, i: (b, i, 0)),
        ],
        out_shape=[
            jax.ShapeDtypeStruct((BATCH, LQ, NH * NCORN), jnp.int32),
            jax.ShapeDtypeStruct((BATCH, LQ, NH * NCORN), jnp.float32),
        ],
        compiler_params=pltpu.CompilerParams(
            dimension_semantics=("parallel", "parallel")),
    )(q_t, jnp.asarray(_POS_T), wofft, bofft, wattnt, battnt, jnp.asarray(_RS))


def _out_proj_body(a_ref, w_ref, b_ref, out_ref):
    out_ref[0] = jnp.dot(a_ref[0], w_ref[...], preferred_element_type=jnp.float32) + b_ref[...]


def _out_proj(attn, w_out, b_out):
    return pl.pallas_call(
        _out_proj_body,
        grid=(BATCH, NQB),
        in_specs=[
            pl.BlockSpec((1, BLKQ, DIM), lambda b, i: (b, i, 0)),
            pl.BlockSpec((DIM, DIM), lambda b, i: (0, 0)),
            pl.BlockSpec((1, DIM), lambda b, i: (0, 0)),
        ],
        out_specs=pl.BlockSpec((1, BLKQ, DIM), lambda b, i: (b, i, 0)),
        out_shape=jax.ShapeDtypeStruct((BATCH, LQ, DIM), jnp.float32),
        compiler_params=pltpu.CompilerParams(
            dimension_semantics=("parallel", "parallel")),
    )(attn, w_out, b_out[None, :])


def _sc_sample_body(table, idx2, wtsf, out, idx_v, wts_v, rb0, rb1, out_v, s0, s1):
    cid = lax.axis_index("c")
    sid = lax.axis_index("s")
    wid = sid * 2 + cid

    def sb_body(sb, carry):
        base_row = pl.multiple_of(wid * RPW + sb * SB, SB)
        pltpu.sync_copy(idx2.at[pl.ds(pl.multiple_of(base_row // CHUNK_ROWS, SB // CHUNK_ROWS), NCH)], idx_v)
        pltpu.sync_copy(wtsf.at[pl.ds(pl.multiple_of(base_row * NCORN, SB * NCORN), SB * NCORN)], wts_v)
        pltpu.async_copy(table.at[idx_v.at[0]], rb0, s0)
        pltpu.async_copy(table.at[idx_v.at[1]], rb1, s1)

        def grp(g, carry2):
            for bi, (rb, sem) in enumerate(((rb0, s0), (rb1, s1))):
                c = g * 2 + bi
                pltpu.make_async_copy(table.at[idx_v.at[c]], rb, sem).wait()
                for r in range(CHUNK_ROWS):
                    acc0a = jnp.zeros((16,), jnp.float32)
                    acc0b = jnp.zeros((16,), jnp.float32)
                    acc1a = jnp.zeros((16,), jnp.float32)
                    acc1b = jnp.zeros((16,), jnp.float32)
                    wrow = [wts_v[pl.ds(c * (CHUNK_ROWS * NCORN) + r * NCORN + t * 16, 16)]
                            for t in range(NCORN // 16)]
                    for j in range(NCORN):
                        k = r * NCORN + j
                        wsp = wrow[j // 16][j % 16]
                        lo = wsp * rb[k, pl.ds(0, 16)]
                        hi = wsp * rb[k, pl.ds(16, 16)]
                        if j % 2 == 0:
                            acc0a = acc0a + lo
                            acc1a = acc1a + hi
                        else:
                            acc0b = acc0b + lo
                            acc1b = acc1b + hi
                    cc = c * CHUNK_ROWS + r
                    out_v[cc, pl.ds(0, 16)] = acc0a + acc0b
                    out_v[cc, pl.ds(16, 16)] = acc1a + acc1b
                nc = c + 2

                @pl.when(nc < NCH)
                def _():
                    pltpu.async_copy(table.at[idx_v.at[nc]], rb, sem)
            return carry2

        lax.fori_loop(0, NCH // 2, grp, 0)
        pltpu.sync_copy(out_v, out.at[pl.ds(pl.multiple_of(base_row, SB), SB)])
        return carry

    lax.fori_loop(0, NSBS, sb_body, 0)


def _sc_sample(table, idx2, wtsf):
    mesh = plsc.VectorSubcoreMesh(core_axis_name="c", subcore_axis_name="s",
                                  num_cores=2, num_subcores=16)
    f = functools.partial(
        pl.kernel,
        out_type=jax.ShapeDtypeStruct((NROWS, DH), jnp.float32),
        mesh=mesh,
        scratch_types=[
            pltpu.VMEM((NCH, CHUNK_ROWS * NCORN), jnp.int32),
            pltpu.VMEM((SB * NCORN,), jnp.float32),
            pltpu.VMEM((CHUNK_ROWS * NCORN, DH), jnp.float32),
            pltpu.VMEM((CHUNK_ROWS * NCORN, DH), jnp.float32),
            pltpu.VMEM((SB, DH), jnp.float32),
            pltpu.SemaphoreType.DMA,
            pltpu.SemaphoreType.DMA,
        ],
        compiler_params=pltpu.CompilerParams(use_tc_tiling_on_sc=False),
    )(_sc_sample_body)
    return f(table, idx2, wtsf)


def kernel(query0, query1, query2, feat0, feat1, feat2, w_value, b_value,
           w_off, b_off, w_attn, b_attn, w_out, b_out):
    feats = [feat0, feat1, feat2]
    querys = [query0, query1, query2]
    inp_t = jnp.concatenate([f.reshape(BATCH, DIM, -1) for f in feats], axis=2)
    q_t = jnp.concatenate([q.reshape(BATCH, DIM, -1) for q in querys], axis=2)
    inp_rows = jnp.transpose(inp_t, (0, 2, 1))

    val = _value_proj(inp_rows, w_value, b_value)
    table = jnp.concatenate([
        jnp.zeros((PAD, DH), jnp.float32),
        val.reshape(NROWS, DH),
        jnp.zeros((PAD, DH), jnp.float32),
    ], axis=0)

    idx, wts = _idx_wts(q_t, w_off, b_off, w_attn, b_attn)
    idx2 = idx.reshape(-1, CHUNK_ROWS * NCORN)
    wtsf = wts.reshape(-1)

    attn_rows = _sc_sample(table, idx2, wtsf)
    attn = attn_rows.reshape(BATCH, LQ, DIM)

    out = _out_proj(attn, w_out, b_out)

    s0, s1 = SIZES[0], SIZES[1]
    f0 = out[:, :s0].reshape(BATCH, SHAPES[0][0], SHAPES[0][1], DIM).transpose(0, 3, 1, 2)
    f1 = out[:, s0:s0 + s1].reshape(BATCH, SHAPES[1][0], SHAPES[1][1], DIM).transpose(0, 3, 1, 2)
    f2 = out[:, s0 + s1:].reshape(BATCH, SHAPES[2][0], SHAPES[2][1], DIM).transpose(0, 3, 1, 2)
    return (f0, f1, f2)
